# baseline (speedup 1.0000x reference)
import functools

import jax
import jax.numpy as jnp
from jax import lax
from jax.experimental import pallas as pl
from jax.experimental.pallas import tpu as pltpu
from jax.experimental.pallas import tpu_sc as plsc

_BATCH = 4096
_FIELDS = 26
_DIM = 64
_NC = 2
_NS = 16
_NW = _NC * _NS
_BPW = _BATCH // _NW
_LANES = 16


def _first_order_body(idx_hbm, w_hbm, out_hbm, idx_v, vals_v, acc_v, sem):
    wid = lax.axis_index("s") * _NC + lax.axis_index("c")
    pltpu.sync_copy(idx_hbm.at[:, pl.ds(wid * _BPW, _BPW)], idx_v)
    copies = [
        pltpu.async_copy(w_hbm.at[idx_v.at[f]], vals_v.at[f], sem)
        for f in range(_FIELDS)
    ]
    for cp in copies:
        cp.wait()
    for c in range(_BPW // _LANES):
        sl = pl.ds(c * _LANES, _LANES)
        acc = vals_v[0, sl]
        for g in range(1, _FIELDS):
            acc = acc + vals_v[g, sl]
        acc_v[sl] = acc
    pltpu.sync_copy(acc_v, out_hbm.at[pl.ds(wid * _BPW, _BPW)])


def _first_order(sparse_inputs, w):
    idx = sparse_inputs.astype(jnp.int32).T
    w_flat = w.reshape(-1)
    mesh = plsc.VectorSubcoreMesh(core_axis_name="c", subcore_axis_name="s")
    k = functools.partial(
        pl.kernel,
        mesh=mesh,
        out_type=jax.ShapeDtypeStruct((_BATCH,), jnp.float32),
        scratch_types=[
            pltpu.VMEM((_FIELDS, _BPW), jnp.int32),
            pltpu.VMEM((_FIELDS, _BPW), jnp.float32),
            pltpu.VMEM((_BPW,), jnp.float32),
            pltpu.SemaphoreType.DMA,
        ],
    )(_first_order_body)
    return k(idx, w_flat)


_FB = 2


def _second_order_body(e_ref, o_ref, s_acc, t2_acc):
    i = pl.program_id(0)
    x = e_ref[...]
    xs = x[0]
    xq = x[0] * x[0]
    for j in range(1, _FB):
        xs = xs + x[j]
        xq = xq + x[j] * x[j]
    t2p = jnp.sum(xq, axis=0)

    @pl.when(i == 0)
    def _init():
        s_acc[...] = xs
        t2_acc[...] = t2p

    @pl.when(i > 0)
    def _accum():
        s_acc[...] = s_acc[...] + xs
        t2_acc[...] = t2_acc[...] + t2p

    @pl.when(i == _FIELDS // _FB - 1)
    def _emit():
        s = s_acc[...]
        o_ref[...] = 0.5 * (jnp.sum(s * s, axis=0) - t2_acc[...])


def _second_order(embed_inputs):
    x = embed_inputs.transpose(1, 2, 0)
    return pl.pallas_call(
        _second_order_body,
        grid=(_FIELDS // _FB,),
        in_specs=[pl.BlockSpec((_FB, _DIM, _BATCH), lambda i: (i, 0, 0))],
        out_specs=pl.BlockSpec((_BATCH,), lambda i: (0,)),
        out_shape=jax.ShapeDtypeStruct((_BATCH,), jnp.float32),
        scratch_shapes=[
            pltpu.VMEM((_DIM, _BATCH), jnp.float32),
            pltpu.VMEM((_BATCH,), jnp.float32),
        ],
    )(x)


def kernel(sparse_inputs, embed_inputs, w):
    first = _first_order(sparse_inputs, w)
    second = _second_order(embed_inputs)
    return (first + second).reshape(_BATCH, 1)


# speedup vs baseline: 1.1307x; 1.1307x over previous
import functools

import jax
import jax.numpy as jnp
from jax import lax
from jax.experimental import pallas as pl
from jax.experimental.pallas import tpu as pltpu
from jax.experimental.pallas import tpu_sc as plsc

_BATCH = 4096
_FIELDS = 26
_DIM = 64
_NC = 2
_NS = 16
_NW = _NC * _NS
_BPW = _BATCH // _NW
_LANES = 16


def _first_order_body(idx_hbm, w_hbm, out_hbm, idx_v, vals_v, acc_v, sem):
    wid = lax.axis_index("s") * _NC + lax.axis_index("c")
    pltpu.sync_copy(idx_hbm.at[:, pl.ds(wid * _BPW, _BPW)], idx_v)
    copies = [
        pltpu.async_copy(w_hbm.at[idx_v.at[f]], vals_v.at[f], sem)
        for f in range(_FIELDS)
    ]
    for cp in copies:
        cp.wait()
    for c in range(_BPW // _LANES):
        sl = pl.ds(c * _LANES, _LANES)
        acc = vals_v[0, sl]
        for g in range(1, _FIELDS):
            acc = acc + vals_v[g, sl]
        acc_v[sl] = acc
    pltpu.sync_copy(acc_v, out_hbm.at[pl.ds(wid * _BPW, _BPW)])


def _first_order(sparse_inputs, w):
    idx = sparse_inputs.astype(jnp.int32).T
    w_flat = w.reshape(-1)
    mesh = plsc.VectorSubcoreMesh(core_axis_name="c", subcore_axis_name="s")
    k = functools.partial(
        pl.kernel,
        mesh=mesh,
        out_type=jax.ShapeDtypeStruct((_BATCH,), jnp.float32),
        scratch_types=[
            pltpu.VMEM((_FIELDS, _BPW), jnp.int32),
            pltpu.VMEM((_FIELDS, _BPW), jnp.float32),
            pltpu.VMEM((_BPW,), jnp.float32),
            pltpu.SemaphoreType.DMA,
        ],
    )(_first_order_body)
    return k(idx, w_flat)


_FB = 2
_NBUF = 4
_NCH = _FIELDS // _FB


def _second_order_body(e_hbm, o_ref, bufs, sems):
    def start(c):
        k = c % _NBUF
        pltpu.make_async_copy(
            e_hbm.at[pl.ds(c * _FB, _FB)], bufs.at[k], sems.at[k]
        ).start()

    for c in range(_NBUF):
        start(c)
    s = None
    t2 = None
    for c in range(_NCH):
        k = c % _NBUF
        pltpu.make_async_copy(
            e_hbm.at[pl.ds(c * _FB, _FB)], bufs.at[k], sems.at[k]
        ).wait()
        x = bufs[k]
        xs = x[0]
        xq = x[0] * x[0]
        for j in range(1, _FB):
            xs = xs + x[j]
            xq = xq + x[j] * x[j]
        t2p = jnp.sum(xq, axis=0)
        s = xs if s is None else s + xs
        t2 = t2p if t2 is None else t2 + t2p
        if c + _NBUF < _NCH:
            start(c + _NBUF)
    o_ref[...] = 0.5 * (jnp.sum(s * s, axis=0) - t2)


def _second_order(embed_inputs):
    x = embed_inputs.transpose(1, 2, 0)
    return pl.pallas_call(
        _second_order_body,
        in_specs=[pl.BlockSpec(memory_space=pl.ANY)],
        out_specs=pl.BlockSpec(memory_space=pltpu.VMEM),
        out_shape=jax.ShapeDtypeStruct((_BATCH,), jnp.float32),
        scratch_shapes=[
            pltpu.VMEM((_NBUF, _FB, _DIM, _BATCH), jnp.float32),
            pltpu.SemaphoreType.DMA((_NBUF,)),
        ],
    )(x)


def kernel(sparse_inputs, embed_inputs, w):
    first = _first_order(sparse_inputs, w)
    second = _second_order(embed_inputs)
    return (first + second).reshape(_BATCH, 1)


# speedup vs baseline: 1.1317x; 1.0009x over previous
import functools

import jax
import jax.numpy as jnp
from jax import lax
from jax.experimental import pallas as pl
from jax.experimental.pallas import tpu as pltpu
from jax.experimental.pallas import tpu_sc as plsc

_BATCH = 4096
_FIELDS = 26
_DIM = 64
_NC = 2
_NS = 16
_NW = _NC * _NS
_BPW = _BATCH // _NW
_LANES = 16


def _first_order_body(idx_hbm, w_hbm, out_hbm, idx_v, vals_v, acc_v, sem):
    wid = lax.axis_index("s") * _NC + lax.axis_index("c")
    pltpu.sync_copy(idx_hbm.at[:, pl.ds(wid * _BPW, _BPW)], idx_v)
    copies = [
        pltpu.async_copy(w_hbm.at[idx_v.at[f]], vals_v.at[f], sem)
        for f in range(_FIELDS)
    ]
    for cp in copies:
        cp.wait()
    for c in range(_BPW // _LANES):
        sl = pl.ds(c * _LANES, _LANES)
        acc = vals_v[0, sl]
        for g in range(1, _FIELDS):
            acc = acc + vals_v[g, sl]
        acc_v[sl] = acc
    pltpu.sync_copy(acc_v, out_hbm.at[pl.ds(wid * _BPW, _BPW)])


def _first_order(sparse_inputs, w):
    idx = sparse_inputs.astype(jnp.int32).T
    w_flat = w.reshape(-1)
    mesh = plsc.VectorSubcoreMesh(core_axis_name="c", subcore_axis_name="s")
    k = functools.partial(
        pl.kernel,
        mesh=mesh,
        out_type=jax.ShapeDtypeStruct((_BATCH,), jnp.float32),
        scratch_types=[
            pltpu.VMEM((_FIELDS, _BPW), jnp.int32),
            pltpu.VMEM((_FIELDS, _BPW), jnp.float32),
            pltpu.VMEM((_BPW,), jnp.float32),
            pltpu.SemaphoreType.DMA,
        ],
    )(_first_order_body)
    return k(idx, w_flat)


_FB = 1
_NBUF = 8
_NCH = _FIELDS // _FB


def _second_order_body(e_hbm, o_ref, bufs, sems):
    def start(c):
        k = c % _NBUF
        pltpu.make_async_copy(
            e_hbm.at[pl.ds(c * _FB, _FB)], bufs.at[k], sems.at[k]
        ).start()

    for c in range(_NBUF):
        start(c)
    s = None
    t2 = None
    for c in range(_NCH):
        k = c % _NBUF
        pltpu.make_async_copy(
            e_hbm.at[pl.ds(c * _FB, _FB)], bufs.at[k], sems.at[k]
        ).wait()
        x = bufs[k]
        xs = x[0]
        xq = x[0] * x[0]
        for j in range(1, _FB):
            xs = xs + x[j]
            xq = xq + x[j] * x[j]
        t2p = jnp.sum(xq, axis=0)
        s = xs if s is None else s + xs
        t2 = t2p if t2 is None else t2 + t2p
        if c + _NBUF < _NCH:
            start(c + _NBUF)
    o_ref[...] = 0.5 * (jnp.sum(s * s, axis=0) - t2)


def _second_order(embed_inputs):
    x = embed_inputs.transpose(1, 2, 0)
    return pl.pallas_call(
        _second_order_body,
        in_specs=[pl.BlockSpec(memory_space=pl.ANY)],
        out_specs=pl.BlockSpec(memory_space=pltpu.VMEM),
        out_shape=jax.ShapeDtypeStruct((_BATCH,), jnp.float32),
        scratch_shapes=[
            pltpu.VMEM((_NBUF, _FB, _DIM, _BATCH), jnp.float32),
            pltpu.SemaphoreType.DMA((_NBUF,)),
        ],
    )(x)


def kernel(sparse_inputs, embed_inputs, w):
    first = _first_order(sparse_inputs, w)
    second = _second_order(embed_inputs)
    return (first + second).reshape(_BATCH, 1)


# speedup vs baseline: 1.2637x; 1.1166x over previous
import functools

import jax
import jax.numpy as jnp
from jax import lax
from jax.experimental import pallas as pl
from jax.experimental.pallas import tpu as pltpu
from jax.experimental.pallas import tpu_sc as plsc

_BATCH = 4096
_FIELDS = 26
_DIM = 64
_NC = 2
_NS = 16
_NW = _NC * _NS
_BPW = _BATCH // _NW
_LANES = 16


def _first_order_body(idx_hbm, w_hbm, out_hbm, idx_v, vals_v, acc_v, w_sp, sem):
    sid = lax.axis_index("s")
    wid = sid * _NC + lax.axis_index("c")

    @pl.when(sid == 0)
    def _stage():
        pltpu.sync_copy(w_hbm, w_sp)

    pltpu.sync_copy(idx_hbm.at[:, pl.ds(wid * _BPW, _BPW)], idx_v)
    plsc.subcore_barrier()
    copies = [
        pltpu.async_copy(w_sp.at[idx_v.at[f]], vals_v.at[f], sem)
        for f in range(_FIELDS)
    ]
    for cp in copies:
        cp.wait()
    for c in range(_BPW // _LANES):
        sl = pl.ds(c * _LANES, _LANES)
        acc = vals_v[0, sl]
        for g in range(1, _FIELDS):
            acc = acc + vals_v[g, sl]
        acc_v[sl] = acc
    pltpu.sync_copy(acc_v, out_hbm.at[pl.ds(wid * _BPW, _BPW)])


def _first_order(sparse_inputs, w):
    idx = sparse_inputs.astype(jnp.int32).T
    w_flat = w.reshape(-1)
    mesh = plsc.VectorSubcoreMesh(core_axis_name="c", subcore_axis_name="s")
    k = functools.partial(
        pl.kernel,
        mesh=mesh,
        out_type=jax.ShapeDtypeStruct((_BATCH,), jnp.float32),
        scratch_types=[
            pltpu.VMEM((_FIELDS, _BPW), jnp.int32),
            pltpu.VMEM((_FIELDS, _BPW), jnp.float32),
            pltpu.VMEM((_BPW,), jnp.float32),
            pltpu.VMEM_SHARED((100000,), jnp.float32),
            pltpu.SemaphoreType.DMA,
        ],
    )(_first_order_body)
    return k(idx, w_flat)


_FB = 1
_NBUF = 8
_NCH = _FIELDS // _FB


def _second_order_body(e_hbm, o_ref, bufs, sems):
    def start(c):
        k = c % _NBUF
        pltpu.make_async_copy(
            e_hbm.at[pl.ds(c * _FB, _FB)], bufs.at[k], sems.at[k]
        ).start()

    for c in range(_NBUF):
        start(c)
    s = None
    t2 = None
    for c in range(_NCH):
        k = c % _NBUF
        pltpu.make_async_copy(
            e_hbm.at[pl.ds(c * _FB, _FB)], bufs.at[k], sems.at[k]
        ).wait()
        x = bufs[k]
        xs = x[0]
        xq = x[0] * x[0]
        for j in range(1, _FB):
            xs = xs + x[j]
            xq = xq + x[j] * x[j]
        t2p = jnp.sum(xq, axis=0)
        s = xs if s is None else s + xs
        t2 = t2p if t2 is None else t2 + t2p
        if c + _NBUF < _NCH:
            start(c + _NBUF)
    o_ref[...] = 0.5 * (jnp.sum(s * s, axis=0) - t2)


def _second_order(embed_inputs):
    x = embed_inputs.transpose(1, 2, 0)
    return pl.pallas_call(
        _second_order_body,
        in_specs=[pl.BlockSpec(memory_space=pl.ANY)],
        out_specs=pl.BlockSpec(memory_space=pltpu.VMEM),
        out_shape=jax.ShapeDtypeStruct((_BATCH,), jnp.float32),
        scratch_shapes=[
            pltpu.VMEM((_NBUF, _FB, _DIM, _BATCH), jnp.float32),
            pltpu.SemaphoreType.DMA((_NBUF,)),
        ],
    )(x)


def kernel(sparse_inputs, embed_inputs, w):
    first = _first_order(sparse_inputs, w)
    second = _second_order(embed_inputs)
    return (first + second).reshape(_BATCH, 1)


# speedup vs baseline: 1.2742x; 1.0083x over previous
import functools

import jax
import jax.numpy as jnp
from jax import lax
from jax.experimental import pallas as pl
from jax.experimental.pallas import tpu as pltpu
from jax.experimental.pallas import tpu_sc as plsc

_BATCH = 4096
_FIELDS = 26
_DIM = 64
_NC = 2
_NS = 16
_NW = _NC * _NS
_BPW = _BATCH // _NW
_LANES = 16


def _first_order_body(idx_hbm, w_hbm, out_hbm, idx_v, vals_v, acc_v, w_sp, sem):
    sid = lax.axis_index("s")
    wid = sid * _NC + lax.axis_index("c")

    @pl.when(sid == 0)
    def _stage():
        pltpu.sync_copy(w_hbm, w_sp)

    pltpu.sync_copy(idx_hbm.at[:, pl.ds(wid * _BPW, _BPW)], idx_v)
    plsc.subcore_barrier()
    copies = [
        pltpu.async_copy(w_sp.at[idx_v.at[f]], vals_v.at[f], sem)
        for f in range(_FIELDS)
    ]
    for cp in copies:
        cp.wait()
    for c in range(_BPW // _LANES):
        sl = pl.ds(c * _LANES, _LANES)
        acc = vals_v[0, sl]
        for g in range(1, _FIELDS):
            acc = acc + vals_v[g, sl]
        acc_v[sl] = acc
    pltpu.sync_copy(acc_v, out_hbm.at[pl.ds(wid * _BPW, _BPW)])


def _first_order(sparse_inputs, w):
    idx = sparse_inputs.astype(jnp.int32).T
    w_flat = w.reshape(-1)
    mesh = plsc.VectorSubcoreMesh(core_axis_name="c", subcore_axis_name="s")
    k = functools.partial(
        pl.kernel,
        mesh=mesh,
        out_type=jax.ShapeDtypeStruct((_BATCH,), jnp.float32),
        scratch_types=[
            pltpu.VMEM((_FIELDS, _BPW), jnp.int32),
            pltpu.VMEM((_FIELDS, _BPW), jnp.float32),
            pltpu.VMEM((_BPW,), jnp.float32),
            pltpu.VMEM_SHARED((w.shape[0],), jnp.float32),
            pltpu.SemaphoreType.DMA,
        ],
    )(_first_order_body)
    return k(idx, w_flat)


_FB = 1
_NBUF = 8
_NCH = _FIELDS // _FB


def _second_order_body(e_hbm, o_ref, bufs, sems):
    def start(c):
        k = c % _NBUF
        pltpu.make_async_copy(
            e_hbm.at[pl.ds(c * _FB, _FB)], bufs.at[k], sems.at[k]
        ).start()

    for c in range(_NBUF):
        start(c)
    s = None
    t2 = None
    for c in range(_NCH):
        k = c % _NBUF
        pltpu.make_async_copy(
            e_hbm.at[pl.ds(c * _FB, _FB)], bufs.at[k], sems.at[k]
        ).wait()
        x = bufs[k]
        xs = x[0]
        xq = x[0] * x[0]
        for j in range(1, _FB):
            xs = xs + x[j]
            xq = xq + x[j] * x[j]
        t2p = jnp.sum(xq, axis=0)
        s = xs if s is None else s + xs
        t2 = t2p if t2 is None else t2 + t2p
        if c + _NBUF < _NCH:
            start(c + _NBUF)
    o_ref[...] = 0.5 * (jnp.sum(s * s, axis=0) - t2)


def _second_order(embed_inputs):
    x = embed_inputs.transpose(1, 2, 0)
    return pl.pallas_call(
        _second_order_body,
        in_specs=[pl.BlockSpec(memory_space=pl.ANY)],
        out_specs=pl.BlockSpec(memory_space=pltpu.VMEM),
        out_shape=jax.ShapeDtypeStruct((_BATCH,), jnp.float32),
        scratch_shapes=[
            pltpu.VMEM((_NBUF, _FB, _DIM, _BATCH), jnp.float32),
            pltpu.SemaphoreType.DMA((_NBUF,)),
        ],
    )(x)


def kernel(sparse_inputs, embed_inputs, w):
    first = _first_order(sparse_inputs, w)
    second = _second_order(embed_inputs)
    return (first + second).reshape(_BATCH, 1)
